# baseline (speedup 1.0000x reference)
import jax
import jax.numpy as jnp
from jax import lax
from jax.experimental import pallas as pl
from jax.experimental.pallas import tpu as pltpu

_EPS = 1e-5
_FP8_MAX = 448.0
_F8 = jnp.float8_e4m3fn

_BM = 1024
_BN = 1024
_CH = 128
_NCH = _BM // _CH


def _norm_chunk(x_ref, r_ref, nw_ref):
    a = x_ref[...] + r_ref[...]
    inv = lax.rsqrt(jnp.mean(a * a, axis=-1, keepdims=True) + _EPS)
    q = jnp.clip(a * inv * nw_ref[...], -_FP8_MAX, _FP8_MAX).astype(_F8)
    return a, q


def _prologue_kernel(x_ref, r_ref, nw_ref, w_ref, add_ref, q0_ref, w8_ref):
    w8_ref[...] = w_ref[...].astype(_F8)

    @pl.when(pl.program_id(0) < _NCH)
    def _():
        a, q = _norm_chunk(x_ref, r_ref, nw_ref)
        add_ref[...] = a
        q0_ref[...] = q


def _fused_kernel(q0_ref, x_ref, r_ref, nw_ref, w_ref, s_ref, add_in_ref,
                  o_ref, add_ref, q_scr):
    del add_in_ref
    i = pl.program_id(0)
    j = pl.program_id(1)

    @pl.when(jnp.logical_and(i == 0, j == 0))
    def _():
        q_scr[0] = q0_ref[...]

    row = jnp.minimum(j, _NCH - 1) * _CH

    def _body(cur, nxt):
        a, q = _norm_chunk(x_ref, r_ref, nw_ref)
        add_ref[...] = a
        q_scr[nxt, pl.ds(row, _CH), :] = q
        acc = lax.dot_general(
            q_scr[cur],
            w_ref[...],
            dimension_numbers=(((1,), (1,)), ((), ())),
            preferred_element_type=jnp.float32,
        )
        o_ref[...] = acc * s_ref[0]

    @pl.when(lax.rem(i, 2) == 0)
    def _():
        _body(0, 1)

    @pl.when(lax.rem(i, 2) == 1)
    def _():
        _body(1, 0)


def kernel(x, residual, norm_weight, w_fp8, input_scale, w_scale):
    n, d_in = x.shape
    d_out = w_fp8.shape[0]

    nw = (norm_weight / input_scale).astype(jnp.float32).reshape(1, d_in)
    out_scale = (input_scale * w_scale).astype(jnp.float32).reshape(1)

    n_blk = n // _BM
    n_ch_total = n // _CH
    bw = 512
    ga = d_out // bw

    add0, q0, w8 = pl.pallas_call(
        _prologue_kernel,
        grid=(ga,),
        in_specs=[
            pl.BlockSpec((_CH, d_in), lambda k: (jnp.minimum(k, _NCH - 1), 0)),
            pl.BlockSpec((_CH, d_in), lambda k: (jnp.minimum(k, _NCH - 1), 0)),
            pl.BlockSpec((1, d_in), lambda k: (0, 0)),
            pl.BlockSpec((bw, d_in), lambda k: (k, 0)),
        ],
        out_specs=[
            pl.BlockSpec((_CH, d_in), lambda k: (jnp.minimum(k, _NCH - 1), 0)),
            pl.BlockSpec((_CH, d_in), lambda k: (jnp.minimum(k, _NCH - 1), 0)),
            pl.BlockSpec((bw, d_in), lambda k: (k, 0)),
        ],
        out_shape=[
            jax.ShapeDtypeStruct((n, d_in), jnp.float32),
            jax.ShapeDtypeStruct((_BM, d_in), _F8),
            jax.ShapeDtypeStruct((d_out, d_in), _F8),
        ],
        compiler_params=pltpu.CompilerParams(
            dimension_semantics=("arbitrary",),
            vmem_limit_bytes=56 * 1024 * 1024,
        ),
    )(x[:_BM], residual[:_BM], nw, w_fp8)

    def _chunk_idx(i, j):
        return (jnp.minimum((i + 1) * _NCH + jnp.minimum(j, _NCH - 1),
                            n_ch_total - 1), 0)

    out, add_out = pl.pallas_call(
        _fused_kernel,
        grid=(n_blk, d_out // _BN),
        in_specs=[
            pl.BlockSpec((_BM, d_in), lambda i, j: (0, 0)),
            pl.BlockSpec((_CH, d_in), _chunk_idx),
            pl.BlockSpec((_CH, d_in), _chunk_idx),
            pl.BlockSpec((1, d_in), lambda i, j: (0, 0)),
            pl.BlockSpec((_BN, d_in), lambda i, j: (j, 0)),
            pl.BlockSpec(memory_space=pltpu.SMEM),
            pl.BlockSpec(memory_space=pl.ANY),
        ],
        out_specs=[
            pl.BlockSpec((_BM, _BN), lambda i, j: (i, j)),
            pl.BlockSpec((_CH, d_in), _chunk_idx),
        ],
        out_shape=[
            jax.ShapeDtypeStruct((n, d_out), jnp.float32),
            jax.ShapeDtypeStruct((n, d_in), jnp.float32),
        ],
        scratch_shapes=[pltpu.VMEM((2, _BM, d_in), _F8)],
        input_output_aliases={6: 1},
        compiler_params=pltpu.CompilerParams(
            dimension_semantics=("arbitrary", "arbitrary"),
            vmem_limit_bytes=56 * 1024 * 1024,
        ),
    )(q0, x, residual, nw, w8, out_scale, add0)

    return (out, add_out)


# speedup vs baseline: 1.0669x; 1.0669x over previous
import jax
import jax.numpy as jnp
from jax import lax
from jax.experimental import pallas as pl
from jax.experimental.pallas import tpu as pltpu

_EPS = 1e-5
_FP8_MAX = 448.0
_F8 = jnp.float8_e4m3fn

_BM = 1024
_BN = 1536
_CH = 128
_NCH = _BM // _CH


def _norm_chunk(x_ref, r_ref, nw_ref):
    a = x_ref[...] + r_ref[...]
    inv = lax.rsqrt(jnp.mean(a * a, axis=-1, keepdims=True) + _EPS)
    q = jnp.clip(a * inv * nw_ref[...], -_FP8_MAX, _FP8_MAX).astype(_F8)
    return a, q


def _prologue_kernel(x_ref, r_ref, nw_ref, w_ref, add_ref, q0_ref, w8_ref):
    w8_ref[...] = w_ref[...].astype(_F8)

    @pl.when(pl.program_id(0) < _NCH)
    def _():
        a, q = _norm_chunk(x_ref, r_ref, nw_ref)
        add_ref[...] = a
        q0_ref[...] = q


def _fused_kernel(q0_ref, x_ref, r_ref, nw_ref, w_ref, s_ref, add_in_ref,
                  o_ref, add_ref, q_scr):
    del add_in_ref
    i = pl.program_id(0)
    j = pl.program_id(1)

    @pl.when(jnp.logical_and(i == 0, j == 0))
    def _():
        q_scr[0] = q0_ref[...]

    row = jnp.minimum(j, _NCH - 1) * _CH

    def _body(cur, nxt):
        a, q = _norm_chunk(x_ref, r_ref, nw_ref)
        add_ref[...] = a
        q_scr[nxt, pl.ds(row, _CH), :] = q
        acc = lax.dot_general(
            q_scr[cur],
            w_ref[...],
            dimension_numbers=(((1,), (1,)), ((), ())),
            preferred_element_type=jnp.float32,
        )
        o_ref[...] = acc * s_ref[0]

    @pl.when(lax.rem(i, 2) == 0)
    def _():
        _body(0, 1)

    @pl.when(lax.rem(i, 2) == 1)
    def _():
        _body(1, 0)


def kernel(x, residual, norm_weight, w_fp8, input_scale, w_scale):
    n, d_in = x.shape
    d_out = w_fp8.shape[0]

    nw = (norm_weight / input_scale).astype(jnp.float32).reshape(1, d_in)
    out_scale = (input_scale * w_scale).astype(jnp.float32).reshape(1)

    n_blk = n // _BM
    n_ch_total = n // _CH
    bw = 512
    ga = d_out // bw

    add0, q0, w8 = pl.pallas_call(
        _prologue_kernel,
        grid=(ga,),
        in_specs=[
            pl.BlockSpec((_CH, d_in), lambda k: (jnp.minimum(k, _NCH - 1), 0)),
            pl.BlockSpec((_CH, d_in), lambda k: (jnp.minimum(k, _NCH - 1), 0)),
            pl.BlockSpec((1, d_in), lambda k: (0, 0)),
            pl.BlockSpec((bw, d_in), lambda k: (k, 0)),
        ],
        out_specs=[
            pl.BlockSpec((_CH, d_in), lambda k: (jnp.minimum(k, _NCH - 1), 0)),
            pl.BlockSpec((_CH, d_in), lambda k: (jnp.minimum(k, _NCH - 1), 0)),
            pl.BlockSpec((bw, d_in), lambda k: (k, 0)),
        ],
        out_shape=[
            jax.ShapeDtypeStruct((n, d_in), jnp.float32),
            jax.ShapeDtypeStruct((_BM, d_in), _F8),
            jax.ShapeDtypeStruct((d_out, d_in), _F8),
        ],
        compiler_params=pltpu.CompilerParams(
            dimension_semantics=("arbitrary",),
            vmem_limit_bytes=56 * 1024 * 1024,
        ),
    )(x[:_BM], residual[:_BM], nw, w_fp8)

    def _chunk_idx(i, j):
        return (jnp.minimum((i + 1) * _NCH + jnp.minimum(j, _NCH - 1),
                            n_ch_total - 1), 0)

    out, add_out = pl.pallas_call(
        _fused_kernel,
        grid=(n_blk, d_out // _BN),
        in_specs=[
            pl.BlockSpec((_BM, d_in), lambda i, j: (0, 0)),
            pl.BlockSpec((_CH, d_in), _chunk_idx),
            pl.BlockSpec((_CH, d_in), _chunk_idx),
            pl.BlockSpec((1, d_in), lambda i, j: (0, 0)),
            pl.BlockSpec((_BN, d_in), lambda i, j: (j, 0)),
            pl.BlockSpec(memory_space=pltpu.SMEM),
            pl.BlockSpec(memory_space=pl.ANY),
        ],
        out_specs=[
            pl.BlockSpec((_BM, _BN), lambda i, j: (i, j)),
            pl.BlockSpec((_CH, d_in), _chunk_idx),
        ],
        out_shape=[
            jax.ShapeDtypeStruct((n, d_out), jnp.float32),
            jax.ShapeDtypeStruct((n, d_in), jnp.float32),
        ],
        scratch_shapes=[pltpu.VMEM((2, _BM, d_in), _F8)],
        input_output_aliases={6: 1},
        compiler_params=pltpu.CompilerParams(
            dimension_semantics=("arbitrary", "arbitrary"),
            vmem_limit_bytes=56 * 1024 * 1024,
        ),
    )(q0, x, residual, nw, w8, out_scale, add0)

    return (out, add_out)


# speedup vs baseline: 1.0756x; 1.0082x over previous
import jax
import jax.numpy as jnp
from jax import lax
from jax.experimental import pallas as pl
from jax.experimental.pallas import tpu as pltpu

_EPS = 1e-5
_FP8_MAX = 448.0
_F8 = jnp.float8_e4m3fn

_BM = 1024
_BN = 1536
_CH = 128
_NCH = _BM // _CH


def _norm_chunk(x_ref, r_ref, nw_ref):
    a = x_ref[...] + r_ref[...]
    inv = lax.rsqrt(jnp.mean(a * a, axis=-1, keepdims=True) + _EPS)
    q = jnp.clip(a * inv * nw_ref[...], -_FP8_MAX, _FP8_MAX).astype(_F8)
    return a, q


def _prologue_kernel(x_ref, r_ref, nw_ref, w_ref, add_ref, q0_ref, w8_ref):
    w8_ref[...] = w_ref[...].astype(_F8)

    @pl.when(pl.program_id(0) < _NCH)
    def _():
        a, q = _norm_chunk(x_ref, r_ref, nw_ref)
        add_ref[...] = a
        q0_ref[...] = q


def _fused_kernel(q0_ref, x_ref, r_ref, nw_ref, w_ref, s_ref, add_in_ref,
                  o_ref, add_ref, q_scr_a, q_scr_b):
    del add_in_ref
    i = pl.program_id(0)
    j = pl.program_id(1)

    @pl.when(jnp.logical_and(i == 0, j == 0))
    def _():
        q_scr_a[...] = q0_ref[...]

    row = jnp.minimum(j, _NCH - 1) * _CH

    def _body(cur_ref, nxt_ref):
        a, q = _norm_chunk(x_ref, r_ref, nw_ref)
        add_ref[...] = a
        nxt_ref[pl.ds(row, _CH), :] = q
        acc = lax.dot_general(
            cur_ref[...],
            w_ref[...],
            dimension_numbers=(((1,), (1,)), ((), ())),
            preferred_element_type=jnp.float32,
        )
        o_ref[...] = acc * s_ref[0]

    @pl.when(lax.rem(i, 2) == 0)
    def _():
        _body(q_scr_a, q_scr_b)

    @pl.when(lax.rem(i, 2) == 1)
    def _():
        _body(q_scr_b, q_scr_a)


def kernel(x, residual, norm_weight, w_fp8, input_scale, w_scale):
    n, d_in = x.shape
    d_out = w_fp8.shape[0]

    nw = (norm_weight / input_scale).astype(jnp.float32).reshape(1, d_in)
    out_scale = (input_scale * w_scale).astype(jnp.float32).reshape(1)

    n_blk = n // _BM
    n_ch_total = n // _CH
    bw = 512
    ga = d_out // bw

    add0, q0, w8 = pl.pallas_call(
        _prologue_kernel,
        grid=(ga,),
        in_specs=[
            pl.BlockSpec((_CH, d_in), lambda k: (jnp.minimum(k, _NCH - 1), 0)),
            pl.BlockSpec((_CH, d_in), lambda k: (jnp.minimum(k, _NCH - 1), 0)),
            pl.BlockSpec((1, d_in), lambda k: (0, 0)),
            pl.BlockSpec((bw, d_in), lambda k: (k, 0)),
        ],
        out_specs=[
            pl.BlockSpec((_CH, d_in), lambda k: (jnp.minimum(k, _NCH - 1), 0)),
            pl.BlockSpec((_CH, d_in), lambda k: (jnp.minimum(k, _NCH - 1), 0)),
            pl.BlockSpec((bw, d_in), lambda k: (k, 0)),
        ],
        out_shape=[
            jax.ShapeDtypeStruct((n, d_in), jnp.float32),
            jax.ShapeDtypeStruct((_BM, d_in), _F8),
            jax.ShapeDtypeStruct((d_out, d_in), _F8),
        ],
        compiler_params=pltpu.CompilerParams(
            dimension_semantics=("arbitrary",),
            vmem_limit_bytes=56 * 1024 * 1024,
        ),
    )(x[:_BM], residual[:_BM], nw, w_fp8)

    def _chunk_idx(i, j):
        return (jnp.minimum((i + 1) * _NCH + jnp.minimum(j, _NCH - 1),
                            n_ch_total - 1), 0)

    out, add_out = pl.pallas_call(
        _fused_kernel,
        grid=(n_blk, d_out // _BN),
        in_specs=[
            pl.BlockSpec((_BM, d_in), lambda i, j: (0, 0)),
            pl.BlockSpec((_CH, d_in), _chunk_idx),
            pl.BlockSpec((_CH, d_in), _chunk_idx),
            pl.BlockSpec((1, d_in), lambda i, j: (0, 0)),
            pl.BlockSpec((_BN, d_in), lambda i, j: (j, 0)),
            pl.BlockSpec(memory_space=pltpu.SMEM),
            pl.BlockSpec(memory_space=pl.ANY),
        ],
        out_specs=[
            pl.BlockSpec((_BM, _BN), lambda i, j: (i, j)),
            pl.BlockSpec((_CH, d_in), _chunk_idx),
        ],
        out_shape=[
            jax.ShapeDtypeStruct((n, d_out), jnp.float32),
            jax.ShapeDtypeStruct((n, d_in), jnp.float32),
        ],
        scratch_shapes=[pltpu.VMEM((_BM, d_in), _F8),
                        pltpu.VMEM((_BM, d_in), _F8)],
        input_output_aliases={6: 1},
        compiler_params=pltpu.CompilerParams(
            dimension_semantics=("arbitrary", "arbitrary"),
            vmem_limit_bytes=56 * 1024 * 1024,
        ),
    )(q0, x, residual, nw, w8, out_scale, add0)

    return (out, add_out)


# speedup vs baseline: 1.0761x; 1.0004x over previous
import jax
import jax.numpy as jnp
from jax import lax
from jax.experimental import pallas as pl
from jax.experimental.pallas import tpu as pltpu

_EPS = 1e-5
_FP8_MAX = 448.0
_F8 = jnp.float8_e4m3fn

_BM = 1024
_BN = 1536
_CH = 128
_NCH = _BM // _CH


def _norm_chunk(x_ref, r_ref, nw_ref):
    a = x_ref[...] + r_ref[...]
    inv = lax.rsqrt(jnp.mean(a * a, axis=-1, keepdims=True) + _EPS)
    q = jnp.clip(a * inv * nw_ref[...], -_FP8_MAX, _FP8_MAX).astype(_F8)
    return a, q


def _prologue_kernel(x_ref, r_ref, nw_ref, w_ref, add_ref, q0_ref, w8_ref):
    w8_ref[...] = w_ref[...].astype(_F8)

    @pl.when(pl.program_id(0) < _NCH)
    def _():
        a, q = _norm_chunk(x_ref, r_ref, nw_ref)
        add_ref[...] = a
        q0_ref[...] = q


def _fused_kernel(q0_ref, x_ref, r_ref, nw_ref, w_ref, s_ref, add_in_ref,
                  o_ref, add_ref, q_scr_a, q_scr_b):
    del add_in_ref
    i = pl.program_id(0)
    j = pl.program_id(1)

    @pl.when(jnp.logical_and(i == 0, j == 0))
    def _():
        q_scr_a[...] = q0_ref[...]

    row = jnp.minimum(j, _NCH - 1) * _CH

    def _body(cur_ref, nxt_ref):
        a, q = _norm_chunk(x_ref, r_ref, nw_ref)
        add_ref[...] = a
        nxt_ref[pl.ds(row, _CH), :] = q
        acc = lax.dot_general(
            cur_ref[...],
            w_ref[...],
            dimension_numbers=(((1,), (1,)), ((), ())),
            preferred_element_type=jnp.float32,
        )
        o_ref[...] = acc * s_ref[0]

    @pl.when(lax.rem(i, 2) == 0)
    def _():
        _body(q_scr_a, q_scr_b)

    @pl.when(lax.rem(i, 2) == 1)
    def _():
        _body(q_scr_b, q_scr_a)


def kernel(x, residual, norm_weight, w_fp8, input_scale, w_scale):
    n, d_in = x.shape
    d_out = w_fp8.shape[0]

    nw = (norm_weight / input_scale).astype(jnp.float32).reshape(1, d_in)
    out_scale = (input_scale * w_scale).astype(jnp.float32).reshape(1)

    n_blk = n // _BM
    n_ch_total = n // _CH
    bw = 1024
    ga = d_out // bw

    add0, q0, w8 = pl.pallas_call(
        _prologue_kernel,
        grid=(ga,),
        in_specs=[
            pl.BlockSpec((_CH, d_in), lambda k: (jnp.minimum(k, _NCH - 1), 0)),
            pl.BlockSpec((_CH, d_in), lambda k: (jnp.minimum(k, _NCH - 1), 0)),
            pl.BlockSpec((1, d_in), lambda k: (0, 0)),
            pl.BlockSpec((bw, d_in), lambda k: (k, 0)),
        ],
        out_specs=[
            pl.BlockSpec((_CH, d_in), lambda k: (jnp.minimum(k, _NCH - 1), 0)),
            pl.BlockSpec((_CH, d_in), lambda k: (jnp.minimum(k, _NCH - 1), 0)),
            pl.BlockSpec((bw, d_in), lambda k: (k, 0)),
        ],
        out_shape=[
            jax.ShapeDtypeStruct((n, d_in), jnp.float32),
            jax.ShapeDtypeStruct((_BM, d_in), _F8),
            jax.ShapeDtypeStruct((d_out, d_in), _F8),
        ],
        compiler_params=pltpu.CompilerParams(
            dimension_semantics=("arbitrary",),
            vmem_limit_bytes=56 * 1024 * 1024,
        ),
    )(x[:_BM], residual[:_BM], nw, w_fp8)

    def _chunk_idx(i, j):
        return (jnp.minimum((i + 1) * _NCH + jnp.minimum(j, _NCH - 1),
                            n_ch_total - 1), 0)

    out, add_out = pl.pallas_call(
        _fused_kernel,
        grid=(n_blk, d_out // _BN),
        in_specs=[
            pl.BlockSpec((_BM, d_in), lambda i, j: (0, 0)),
            pl.BlockSpec((_CH, d_in), _chunk_idx),
            pl.BlockSpec((_CH, d_in), _chunk_idx),
            pl.BlockSpec((1, d_in), lambda i, j: (0, 0)),
            pl.BlockSpec((_BN, d_in), lambda i, j: (j, 0)),
            pl.BlockSpec(memory_space=pltpu.SMEM),
            pl.BlockSpec(memory_space=pl.ANY),
        ],
        out_specs=[
            pl.BlockSpec((_BM, _BN), lambda i, j: (i, j)),
            pl.BlockSpec((_CH, d_in), _chunk_idx),
        ],
        out_shape=[
            jax.ShapeDtypeStruct((n, d_out), jnp.float32),
            jax.ShapeDtypeStruct((n, d_in), jnp.float32),
        ],
        scratch_shapes=[pltpu.VMEM((_BM, d_in), _F8),
                        pltpu.VMEM((_BM, d_in), _F8)],
        input_output_aliases={6: 1},
        compiler_params=pltpu.CompilerParams(
            dimension_semantics=("arbitrary", "arbitrary"),
            vmem_limit_bytes=56 * 1024 * 1024,
        ),
    )(q0, x, residual, nw, w8, out_scale, add0)

    return (out, add_out)


# speedup vs baseline: 1.1220x; 1.0426x over previous
import jax
import jax.numpy as jnp
from jax import lax
from jax.experimental import pallas as pl
from jax.experimental.pallas import tpu as pltpu

_EPS = 1e-5
_FP8_MAX = 448.0
_F8 = jnp.float8_e4m3fn

_BM = 1024
_BN = 1536
_CH = 128
_NCH = _BM // _CH


def _norm_chunk(x_ref, r_ref, nw_ref):
    a = x_ref[...] + r_ref[...]
    inv = lax.rsqrt(jnp.mean(a * a, axis=-1, keepdims=True) + _EPS)
    q = jnp.clip(a * inv * nw_ref[...], -_FP8_MAX, _FP8_MAX).astype(_F8)
    return a, q


def _prologue_kernel(x_ref, r_ref, nw_ref, w_ref, add_ref, q0_ref, w8_ref):
    w8_ref[...] = w_ref[...].astype(_F8)

    @pl.when(pl.program_id(0) < _NCH)
    def _():
        a, q = _norm_chunk(x_ref, r_ref, nw_ref)
        add_ref[...] = a
        q0_ref[...] = q


def _fused_kernel(q0_ref, x_ref, r_ref, nw_ref, w_ref, s_ref, add_in_ref,
                  o_ref, add_ref, q_scr_a, q_scr_b):
    del add_in_ref
    i = pl.program_id(0)
    j = pl.program_id(1)

    @pl.when(jnp.logical_and(i == 0, j == 0))
    def _():
        q_scr_a[...] = q0_ref[...]

    row = jnp.minimum(j, _NCH - 1) * _CH

    def _body(cur_ref, nxt_ref):
        a, q = _norm_chunk(x_ref, r_ref, nw_ref)
        add_ref[...] = a
        nxt_ref[pl.ds(row, _CH), :] = q
        acc = lax.dot_general(
            cur_ref[...],
            w_ref[...],
            dimension_numbers=(((1,), (1,)), ((), ())),
            preferred_element_type=jnp.float32,
        )
        o_ref[...] = acc * s_ref[0]

    @pl.when(lax.rem(i, 2) == 0)
    def _():
        _body(q_scr_a, q_scr_b)

    @pl.when(lax.rem(i, 2) == 1)
    def _():
        _body(q_scr_b, q_scr_a)


def kernel(x, residual, norm_weight, w_fp8, input_scale, w_scale):
    n, d_in = x.shape
    d_out = w_fp8.shape[0]

    nw = (norm_weight / input_scale).astype(jnp.float32).reshape(1, d_in)
    out_scale = (input_scale * w_scale).astype(jnp.float32).reshape(1)

    n_blk = n // _BM
    n_ch_total = n // _CH
    bw = 1024
    ga = d_out // bw

    add0, q0, w8 = pl.pallas_call(
        _prologue_kernel,
        grid=(ga,),
        in_specs=[
            pl.BlockSpec((_CH, d_in), lambda k: (jnp.minimum(k, _NCH - 1), 0)),
            pl.BlockSpec((_CH, d_in), lambda k: (jnp.minimum(k, _NCH - 1), 0)),
            pl.BlockSpec((1, d_in), lambda k: (0, 0)),
            pl.BlockSpec((bw, d_in), lambda k: (k, 0)),
        ],
        out_specs=[
            pl.BlockSpec((_CH, d_in), lambda k: (jnp.minimum(k, _NCH - 1), 0)),
            pl.BlockSpec((_CH, d_in), lambda k: (jnp.minimum(k, _NCH - 1), 0)),
            pl.BlockSpec((bw, d_in), lambda k: (k, 0)),
        ],
        out_shape=[
            jax.ShapeDtypeStruct((n, d_in), jnp.float32),
            jax.ShapeDtypeStruct((_BM, d_in), _F8),
            jax.ShapeDtypeStruct((d_out, d_in), _F8),
        ],
        compiler_params=pltpu.CompilerParams(
            dimension_semantics=("arbitrary",),
            vmem_limit_bytes=56 * 1024 * 1024,
        ),
    )(x, residual, nw, w_fp8)

    def _chunk_idx(i, j):
        return (jnp.minimum((i + 1) * _NCH + jnp.minimum(j, _NCH - 1),
                            n_ch_total - 1), 0)

    out, add_out = pl.pallas_call(
        _fused_kernel,
        grid=(n_blk, d_out // _BN),
        in_specs=[
            pl.BlockSpec((_BM, d_in), lambda i, j: (0, 0)),
            pl.BlockSpec((_CH, d_in), _chunk_idx),
            pl.BlockSpec((_CH, d_in), _chunk_idx),
            pl.BlockSpec((1, d_in), lambda i, j: (0, 0)),
            pl.BlockSpec((_BN, d_in), lambda i, j: (j, 0)),
            pl.BlockSpec(memory_space=pltpu.SMEM),
            pl.BlockSpec(memory_space=pl.ANY),
        ],
        out_specs=[
            pl.BlockSpec((_BM, _BN), lambda i, j: (i, j)),
            pl.BlockSpec((_CH, d_in), _chunk_idx),
        ],
        out_shape=[
            jax.ShapeDtypeStruct((n, d_out), jnp.float32),
            jax.ShapeDtypeStruct((n, d_in), jnp.float32),
        ],
        scratch_shapes=[pltpu.VMEM((_BM, d_in), _F8),
                        pltpu.VMEM((_BM, d_in), _F8)],
        input_output_aliases={6: 1},
        compiler_params=pltpu.CompilerParams(
            dimension_semantics=("arbitrary", "arbitrary"),
            vmem_limit_bytes=56 * 1024 * 1024,
        ),
    )(q0, x, residual, nw, w8, out_scale, add0)

    return (out, add_out)
